# baseline (speedup 1.0000x reference)
import functools

import jax
import jax.numpy as jnp
import numpy as np
from jax import lax
from jax.experimental import pallas as pl
from jax.experimental.pallas import tpu as pltpu
from jax.experimental.pallas import tpu_sc as plsc

N = 10000
E = 160000
F = 256
H = 256
R = 8
G = 64
BI = 400

NC, NS = 2, 16
NW = NC * NS
EPW = 5008
EPAD = NW * EPW
_MESH = plsc.VectorSubcoreMesh(core_axis_name="c", subcore_axis_name="s",
                               num_cores=NC, num_subcores=NS)
_SC_PARAMS = pltpu.CompilerParams(needs_layout_passes=False)


KROWS = 640
KDUMP = KROWS - 1


NK = KROWS * 128


def _cnt_body(dst_hbm, type_hbm, out_hbm, dbuf, tbuf, hist):
    c = lax.axis_index("c")
    s = lax.axis_index("s")
    w = c * NS + s

    def zslice(j, _):
        hist[pl.ds(j * 16, 16)] = jnp.zeros((16,), jnp.float32)
        return 0
    lax.fori_loop(0, NK // 16, zslice, 0)

    base = w * EPW
    pltpu.sync_copy(dst_hbm.at[pl.ds(base, EPW)], dbuf)
    pltpu.sync_copy(type_hbm.at[pl.ds(base, EPW)], tbuf)

    iota16 = jnp.arange(16, dtype=jnp.int32)
    ones16 = jnp.full((16,), 1.0, jnp.float32)

    def chunk(j, _):
        d16 = dbuf[pl.ds(j * 16, 16)]
        t16 = tbuf[pl.ds(j * 16, 16)]
        key = d16 * R + t16
        for l in range(16):
            plsc.addupdate_scatter(hist, [key], ones16, mask=iota16 == l)
        return 0

    lax.fori_loop(0, EPW // 16, chunk, 0)
    pltpu.sync_copy(hist, out_hbm.at[w])


def _cnt_sc(dst_pad, type_pad):
    return pl.kernel(
        _cnt_body,
        out_type=jax.ShapeDtypeStruct((NW, NK), jnp.float32),
        mesh=_MESH,
        scratch_types=[
            pltpu.VMEM((EPW,), jnp.int32),
            pltpu.VMEM((EPW,), jnp.int32),
            pltpu.VMEM((NK,), jnp.float32),
        ],
        compiler_params=_SC_PARAMS,
    )(dst_pad, type_pad)


def _inv_body(cnt_ref, inv_ref):
    tot = jnp.sum(cnt_ref[...], axis=0)
    inv_ref[...] = 1.0 / jnp.maximum(tot, 1.0)


def _inv_tc(cnt_part):
    blk = 8192
    return pl.pallas_call(
        _inv_body,
        grid=(NK // blk,),
        in_specs=[pl.BlockSpec((NW, blk), lambda i: (0, i))],
        out_specs=pl.BlockSpec((blk,), lambda i: (i,)),
        out_shape=jax.ShapeDtypeStruct((NK,), jnp.float32),
    )(cnt_part)


NB = N // BI

_t = np.arange(256)
_p = _t % 128
_PVEC = (_t // 128) * 128 + (_p // 32) * 32 + (_p % 2) * 16 + (_p % 32) // 2


def _mm_body(x_ref, w_ref, hh_ref):
    xb = x_ref[...].astype(jnp.bfloat16)
    for k in range(9):
        res = jnp.dot(xb, w_ref[k], preferred_element_type=jnp.float32)
        hh_ref[0, k, 0] = res[:, :128]
        hh_ref[0, k, 1] = res[:, 128:]


def _mm_all(x, w_cat):
    return pl.pallas_call(
        _mm_body,
        grid=(NB,),
        in_specs=[
            pl.BlockSpec((BI, F), lambda i: (i, 0)),
            pl.BlockSpec((9, F, H), lambda i: (0, 0, 0)),
        ],
        out_specs=pl.BlockSpec((1, 9, 2, BI, 128), lambda i: (i, 0, 0, 0, 0)),
        out_shape=jax.ShapeDtypeStruct((NB, 9, 2, BI, 128), jnp.float32),
    )(x, w_cat)


EPT = E // NS
NSEG = 5
SEGE = EPT // NSEG
KCH = 80
NCH = SEGE // KCH


def _agg_body(src_hbm, dst_hbm, type_hbm, inv_hbm, hh_hbm, out_hbm,
              sbuf, tbuf, dbuf, gseg, dseg, kseg, nbuf, rows, zb,
              sem_g, sem_n, sem_s, acc_sh):
    c = lax.axis_index("c")
    s = lax.axis_index("s")
    iota16 = jnp.arange(16, dtype=jnp.int32)

    for j in range(8 * 8):
        zb[j // 8, pl.ds((j % 8) * 16, 16)] = jnp.zeros((16,), jnp.float32)

    def zc(i, _):
        pltpu.sync_copy(zb, acc_sh.at[pl.ds(s * 624 + i * 8, 8)])
        return 0
    lax.fori_loop(0, 78, zc, 0)
    @pl.when(s == NS - 1)
    def _():
        pltpu.sync_copy(zb, acc_sh.at[pl.ds(9984, 8)])
        pltpu.sync_copy(zb, acc_sh.at[pl.ds(9992, 8)])
    plsc.subcore_barrier()

    base = s * EPT
    for seg in range(NSEG):
        sb = base + seg * SEGE
        pltpu.sync_copy(src_hbm.at[pl.ds(sb, SEGE)], sbuf)
        pltpu.sync_copy(dst_hbm.at[pl.ds(sb, SEGE)], dbuf)
        pltpu.sync_copy(type_hbm.at[pl.ds(sb, SEGE)], tbuf)

        def prep(q, _):
            cc = q // (KCH // 16)
            qq = q % (KCH // 16)
            sl = pl.ds(q * 16, 16)
            s16 = sbuf[sl]
            t16 = tbuf[sl]
            d16 = dbuf[sl]
            gi = (((s16 // BI) * 9 + t16) * 2 + c) * BI + s16 % BI
            gseg[cc, pl.ds(qq * 16, 16)] = gi
            dseg[cc, pl.ds(qq * 16, 16)] = d16
            kseg[cc, pl.ds(qq * 16, 16)] = d16 * R + t16
            return 0
        lax.fori_loop(0, SEGE // 16, prep, 0)

        for p in range(2):
            pltpu.async_copy(hh_hbm.at[gseg.at[p]], rows.at[p], sem_g)
            pltpu.async_copy(inv_hbm.at[kseg.at[p]],
                             nbuf.at[pl.ds(p * KCH, KCH)], sem_n)

        def chunk(cc, _):
            pltpu.make_async_copy(hh_hbm.at[gseg.at[cc]],
                                  rows.at[cc % 3], sem_g).wait()
            pltpu.make_async_copy(inv_hbm.at[kseg.at[cc]],
                                  nbuf.at[pl.ds((cc % 3) * KCH, KCH)],
                                  sem_n).wait()

            def scale(e, _):
                sp = plsc.load_gather(nbuf, [jnp.full((16,), 0, jnp.int32)
                                             + ((cc % 3) * KCH + e)])
                for q in range(8):
                    sl = pl.ds(q * 16, 16)
                    rows[cc % 3, e, sl] = rows[cc % 3, e, sl] * sp
                return 0
            lax.fori_loop(0, KCH, scale, 0)
            @pl.when(cc > 0)
            def _():
                pltpu.make_async_copy(rows.at[(cc + 2) % 3],
                                      acc_sh.at[dseg.at[cc - 1]], sem_s).wait()
            @pl.when(cc < NCH - 2)
            def _():
                nb = ((cc + 2) % 3) * KCH
                pltpu.async_copy(hh_hbm.at[gseg.at[cc + 2]],
                                 rows.at[(cc + 2) % 3], sem_g)
                pltpu.async_copy(inv_hbm.at[kseg.at[cc + 2]],
                                 nbuf.at[pl.ds(nb, KCH)], sem_n)
            pltpu.async_copy(rows.at[cc % 3], acc_sh.at[dseg.at[cc]], sem_s,
                             add=True)
            return 0
        lax.fori_loop(0, NCH, chunk, 0)
        pltpu.make_async_copy(rows.at[(NCH - 1) % 3],
                              acc_sh.at[dseg.at[NCH - 1]], sem_s).wait()

    plsc.subcore_barrier()
    pltpu.sync_copy(acc_sh.at[pl.ds(s * 624, 624)],
                    out_hbm.at[c, pl.ds(s * 624, 624)])
    @pl.when(s == NS - 1)
    def _():
        pltpu.sync_copy(acc_sh.at[pl.ds(9984, 16)],
                        out_hbm.at[c, pl.ds(9984, 16)])


def _agg_sc(src, dst, etype, inv_flat, hh_flat):
    return pl.kernel(
        _agg_body,
        out_type=jax.ShapeDtypeStruct((NC, N, 128), jnp.float32),
        mesh=_MESH,
        scratch_types=[
            pltpu.VMEM((SEGE,), jnp.int32),
            pltpu.VMEM((SEGE,), jnp.int32),
            pltpu.VMEM((SEGE,), jnp.int32),
            pltpu.VMEM((NCH, KCH), jnp.int32),
            pltpu.VMEM((NCH, KCH), jnp.int32),
            pltpu.VMEM((NCH, KCH), jnp.int32),
            pltpu.VMEM((3 * KCH,), jnp.float32),
            pltpu.VMEM((3, KCH, 128), jnp.float32),
            pltpu.VMEM((8, 128), jnp.float32),
            pltpu.SemaphoreType.DMA,
            pltpu.SemaphoreType.DMA,
            pltpu.SemaphoreType.DMA,
            pltpu.VMEM_SHARED((N, 128), jnp.float32),
        ],
        compiler_params=_SC_PARAMS,
    )(src, dst, etype, inv_flat, hh_flat)


def _epi_body(hh_ref, ctr_ref, b_ref, h_ref):
    root = jnp.concatenate([hh_ref[0, 0, 0], hh_ref[0, 0, 1]],
                           axis=-1).astype(jnp.float32)
    ctr = jnp.concatenate([ctr_ref[0], ctr_ref[1]], axis=-1)
    h_ref[...] = jax.nn.relu(root + ctr + b_ref[...])


def _epi_tc(hh, contrib, b):
    return pl.pallas_call(
        _epi_body,
        grid=(NB,),
        in_specs=[
            pl.BlockSpec((1, 1, 2, BI, 128), lambda i: (i, R, 0, 0, 0)),
            pl.BlockSpec((2, BI, 128), lambda i: (0, i, 0)),
            pl.BlockSpec((1, H), lambda i: (0, 0)),
        ],
        out_specs=pl.BlockSpec((BI, H), lambda i: (i, 0)),
        out_shape=jax.ShapeDtypeStruct((N, H), jnp.float32),
    )(hh, contrib, b)


def _ss_body(b_ref, ss_ref):
    b = b_ref[...]
    g_ids = lax.broadcasted_iota(jnp.int32, (128, 1), 0)
    lt = (b < g_ids).astype(jnp.int32)
    ss_ref[...] = jnp.sum(lt, axis=1)


NPADB = 10240


def _ss_tc(batch_pad):
    return pl.pallas_call(
        _ss_body,
        in_specs=[pl.BlockSpec((1, NPADB), lambda: (0, 0))],
        out_specs=pl.BlockSpec((128,), lambda: (0,)),
        out_shape=jax.ShapeDtypeStruct((128,), jnp.int32),
    )(batch_pad)


def _pool_body(h_hbm, ss_hbm, out_hbm, ssv, rowbuf, acc):
    c = lax.axis_index("c")
    s = lax.axis_index("s")
    w = c * NS + s
    iota16 = jnp.arange(16, dtype=jnp.int32)
    ninf = jnp.full((16,), -jnp.inf, jnp.float32)
    for q in range(32):
        acc[q // 16, pl.ds((q % 16) * 16, 16)] = ninf
    pltpu.sync_copy(ss_hbm, ssv)

    for gg in range(2):
        g = 2 * w + gg
        ssl = ssv[pl.ds((g // 16) * 16, 16)]
        s0 = jnp.max(jnp.where(iota16 == g % 16, ssl, -1))
        gn = g + 1
        ssl2 = ssv[pl.ds((gn // 16) * 16, 16)]
        s1 = jnp.max(jnp.where(iota16 == gn % 16, ssl2, -1))
        ro0 = (s0 // 16) * 16
        nch = (s1 - ro0 + 15) // 16

        def chunk(ch, _):
            ro = jnp.minimum(ro0 + ch * 16, N - 16)
            pltpu.sync_copy(h_hbm.at[pl.ds(ro, 16)], rowbuf)

            def row(j, _):
                @pl.when(jnp.logical_and(ro + j >= s0, ro + j < s1))
                def _():
                    for q in range(16):
                        sl = pl.ds(q * 16, 16)
                        acc[gg, sl] = jnp.maximum(acc[gg, sl], rowbuf[j, sl])
                return 0
            lax.fori_loop(0, 16, row, 0)
            return 0
        lax.fori_loop(0, nch, chunk, 0)

    pltpu.sync_copy(acc, out_hbm.at[w])


def _pool_sc(h, ss):
    return pl.kernel(
        _pool_body,
        out_type=jax.ShapeDtypeStruct((NW, 2, H), jnp.float32),
        mesh=_MESH,
        scratch_types=[
            pltpu.VMEM((128,), jnp.int32),
            pltpu.VMEM((16, H), jnp.float32),
            pltpu.VMEM((2, H), jnp.float32),
        ],
        compiler_params=_SC_PARAMS,
    )(h, ss)


def _layer(x, src, dst, edge_type, inv_flat, w_cat, b):
    hh = _mm_all(x, w_cat)
    hh_flat = hh.reshape(NB * 9 * 2 * BI, 128)
    contrib = _agg_sc(src, dst, edge_type, inv_flat, hh_flat)
    return _epi_tc(hh, contrib, b.reshape(1, H))


def kernel(x, edge_index, edge_type, batch, Wr1, Wroot1, b1, Wr2, Wroot2, b2):
    src = edge_index[0]
    dst = edge_index[1]
    npad = EPAD - E
    dst_pad = jnp.concatenate([dst, jnp.full((npad,), N, jnp.int32)])
    type_pad = jnp.concatenate([edge_type, jnp.zeros((npad,), jnp.int32)])
    cnt_part = _cnt_sc(dst_pad, type_pad)
    inv_flat = _inv_tc(cnt_part)

    w_cat1 = jnp.concatenate([Wr1, Wroot1[None]], axis=0).astype(jnp.bfloat16)
    w_cat2 = jnp.concatenate([Wr2, Wroot2[None]], axis=0).astype(jnp.bfloat16)
    h1 = _layer(x, src, dst, edge_type, inv_flat, w_cat1, b1)
    h2 = _layer(h1, src, dst, edge_type, inv_flat, w_cat2, b2)

    batch_pad = jnp.concatenate(
        [batch, jnp.full((NPADB - N,), G, jnp.int32)]).reshape(1, NPADB)
    ss = _ss_tc(batch_pad)
    pooled = _pool_sc(h2, ss).reshape(G, H)
    return (h2, pooled)


# speedup vs baseline: 1.8057x; 1.8057x over previous
import functools

import jax
import jax.numpy as jnp
import numpy as np
from jax import lax
from jax.experimental import pallas as pl
from jax.experimental.pallas import tpu as pltpu
from jax.experimental.pallas import tpu_sc as plsc

N = 10000
E = 160000
F = 256
H = 256
R = 8
G = 64
BI = 400

NC, NS = 2, 16
NW = NC * NS
EPW = 5008
EPAD = NW * EPW
_MESH = plsc.VectorSubcoreMesh(core_axis_name="c", subcore_axis_name="s",
                               num_cores=NC, num_subcores=NS)
_SC_PARAMS = pltpu.CompilerParams(needs_layout_passes=False)


KROWS = 640
KDUMP = KROWS - 1


NK = KROWS * 128


def _cnt_body(dst_hbm, type_hbm, out_hbm, dbuf, tbuf, hist):
    c = lax.axis_index("c")
    s = lax.axis_index("s")
    w = c * NS + s

    def zslice(j, _):
        hist[pl.ds(j * 16, 16)] = jnp.zeros((16,), jnp.float32)
        return 0
    lax.fori_loop(0, NK // 16, zslice, 0)

    base = w * EPW
    pltpu.sync_copy(dst_hbm.at[pl.ds(base, EPW)], dbuf)
    pltpu.sync_copy(type_hbm.at[pl.ds(base, EPW)], tbuf)

    iota16 = jnp.arange(16, dtype=jnp.int32)
    ones16 = jnp.full((16,), 1.0, jnp.float32)

    def chunk(j, _):
        d16 = dbuf[pl.ds(j * 16, 16)]
        t16 = tbuf[pl.ds(j * 16, 16)]
        key = d16 * R + t16
        for l in range(16):
            plsc.addupdate_scatter(hist, [key], ones16, mask=iota16 == l)
        return 0

    lax.fori_loop(0, EPW // 16, chunk, 0)
    pltpu.sync_copy(hist, out_hbm.at[w])


def _cnt_sc(dst_pad, type_pad):
    return pl.kernel(
        _cnt_body,
        out_type=jax.ShapeDtypeStruct((NW, NK), jnp.float32),
        mesh=_MESH,
        scratch_types=[
            pltpu.VMEM((EPW,), jnp.int32),
            pltpu.VMEM((EPW,), jnp.int32),
            pltpu.VMEM((NK,), jnp.float32),
        ],
        compiler_params=_SC_PARAMS,
    )(dst_pad, type_pad)


def _inv_body(cnt_ref, inv_ref):
    tot = jnp.sum(cnt_ref[...], axis=0)
    inv_ref[...] = 1.0 / jnp.maximum(tot, 1.0)


def _inv_tc(cnt_part):
    blk = 8192
    return pl.pallas_call(
        _inv_body,
        grid=(NK // blk,),
        in_specs=[pl.BlockSpec((NW, blk), lambda i: (0, i))],
        out_specs=pl.BlockSpec((blk,), lambda i: (i,)),
        out_shape=jax.ShapeDtypeStruct((NK,), jnp.float32),
    )(cnt_part)


NB = N // BI

_t = np.arange(256)
_p = _t % 128
_PVEC = (_t // 128) * 128 + (_p // 32) * 32 + (_p % 2) * 16 + (_p % 32) // 2


def _mm_body(x_ref, w_ref, hh_ref):
    xb = x_ref[...].astype(jnp.bfloat16)
    for k in range(9):
        res = jnp.dot(xb, w_ref[k], preferred_element_type=jnp.float32)
        hh_ref[0, k, 0] = res[:, :128]
        hh_ref[0, k, 1] = res[:, 128:]


def _mm_all(x, w_cat):
    return pl.pallas_call(
        _mm_body,
        grid=(NB,),
        in_specs=[
            pl.BlockSpec((BI, F), lambda i: (i, 0)),
            pl.BlockSpec((9, F, H), lambda i: (0, 0, 0)),
        ],
        out_specs=pl.BlockSpec((1, 9, 2, BI, 128), lambda i: (i, 0, 0, 0, 0)),
        out_shape=jax.ShapeDtypeStruct((NB, 9, 2, BI, 128), jnp.float32),
    )(x, w_cat)


EPT = E // NS
NSEG = 5
SEGE = EPT // NSEG
KCH = 80
NCH = SEGE // KCH


def _agg_body(src_hbm, dst_hbm, type_hbm, inv_hbm, hh_hbm, out_hbm,
              sbuf, tbuf, dbuf, gseg, dseg, kseg, nbuf, rows, zb,
              sem_g, sem_n, sem_s, acc_sh):
    c = lax.axis_index("c")
    s = lax.axis_index("s")
    iota16 = jnp.arange(16, dtype=jnp.int32)

    for j in range(16 * 8):
        zb[j // 8, pl.ds((j % 8) * 16, 16)] = jnp.zeros((16,), jnp.float32)
    for i in range(39):
        pltpu.sync_copy(zb, acc_sh.at[pl.ds(s * 624 + i * 16, 16)])
    @pl.when(s == NS - 1)
    def _():
        pltpu.sync_copy(zb, acc_sh.at[pl.ds(9984, 16)])
    plsc.subcore_barrier()

    base = s * EPT
    for seg in range(NSEG):
        sb = base + seg * SEGE
        pltpu.sync_copy(src_hbm.at[pl.ds(sb, SEGE)], sbuf)
        pltpu.sync_copy(dst_hbm.at[pl.ds(sb, SEGE)], dbuf)
        pltpu.sync_copy(type_hbm.at[pl.ds(sb, SEGE)], tbuf)

        def prep(q, _):
            cc = q // (KCH // 16)
            qq = q % (KCH // 16)
            sl = pl.ds(q * 16, 16)
            s16 = sbuf[sl]
            t16 = tbuf[sl]
            d16 = dbuf[sl]
            gi = (((s16 // BI) * 9 + t16) * 2 + c) * BI + s16 % BI
            gseg[cc, pl.ds(qq * 16, 16)] = gi
            dseg[cc, pl.ds(qq * 16, 16)] = d16
            kseg[cc, pl.ds(qq * 16, 16)] = d16 * R + t16
            return 0
        lax.fori_loop(0, SEGE // 16, prep, 0)

        pltpu.async_copy(hh_hbm.at[gseg.at[0]], rows.at[0], sem_g)
        pltpu.async_copy(inv_hbm.at[kseg.at[0]], nbuf.at[pl.ds(0, KCH)],
                         sem_n)

        def chunk(cc, _):
            @pl.when(cc > 0)
            def _():
                pltpu.make_async_copy(rows.at[(cc + 1) % 2],
                                      acc_sh.at[dseg.at[cc - 1]], sem_s).wait()
            @pl.when(cc < NCH - 1)
            def _():
                nb = ((cc + 1) % 2) * KCH
                pltpu.async_copy(hh_hbm.at[gseg.at[cc + 1]],
                                 rows.at[(cc + 1) % 2], sem_g)
                pltpu.async_copy(inv_hbm.at[kseg.at[cc + 1]],
                                 nbuf.at[pl.ds(nb, KCH)], sem_n)
            pltpu.make_async_copy(hh_hbm.at[gseg.at[cc]],
                                  rows.at[cc % 2], sem_g).wait()
            pltpu.make_async_copy(inv_hbm.at[kseg.at[cc]],
                                  nbuf.at[pl.ds((cc % 2) * KCH, KCH)],
                                  sem_n).wait()

            def scale(e, _):
                sp = plsc.load_gather(nbuf, [jnp.full((16,), 0, jnp.int32)
                                             + ((cc % 2) * KCH + e)])
                for q in range(8):
                    sl = pl.ds(q * 16, 16)
                    rows[cc % 2, e, sl] = rows[cc % 2, e, sl] * sp
                return 0
            lax.fori_loop(0, KCH, scale, 0)
            pltpu.async_copy(rows.at[cc % 2], acc_sh.at[dseg.at[cc]], sem_s,
                             add=True)
            return 0
        lax.fori_loop(0, NCH, chunk, 0)
        pltpu.make_async_copy(rows.at[(NCH - 1) % 2],
                              acc_sh.at[dseg.at[NCH - 1]], sem_s).wait()

    plsc.subcore_barrier()
    pltpu.sync_copy(acc_sh.at[pl.ds(s * 624, 624)],
                    out_hbm.at[c, pl.ds(s * 624, 624)])
    @pl.when(s == NS - 1)
    def _():
        pltpu.sync_copy(acc_sh.at[pl.ds(9984, 16)],
                        out_hbm.at[c, pl.ds(9984, 16)])


def _agg_sc(src, dst, etype, inv_flat, hh_flat):
    return pl.kernel(
        _agg_body,
        out_type=jax.ShapeDtypeStruct((NC, N, 128), jnp.float32),
        mesh=_MESH,
        scratch_types=[
            pltpu.VMEM((SEGE,), jnp.int32),
            pltpu.VMEM((SEGE,), jnp.int32),
            pltpu.VMEM((SEGE,), jnp.int32),
            pltpu.VMEM((NCH, KCH), jnp.int32),
            pltpu.VMEM((NCH, KCH), jnp.int32),
            pltpu.VMEM((NCH, KCH), jnp.int32),
            pltpu.VMEM((2 * KCH,), jnp.float32),
            pltpu.VMEM((2, KCH, 128), jnp.float32),
            pltpu.VMEM((16, 128), jnp.float32),
            pltpu.SemaphoreType.DMA,
            pltpu.SemaphoreType.DMA,
            pltpu.SemaphoreType.DMA,
            pltpu.VMEM_SHARED((N, 128), jnp.float32),
        ],
        compiler_params=_SC_PARAMS,
    )(src, dst, etype, inv_flat, hh_flat)


def _epi_body(hh_ref, ctr_ref, b_ref, h_ref):
    root = jnp.concatenate([hh_ref[0, 0, 0], hh_ref[0, 0, 1]],
                           axis=-1).astype(jnp.float32)
    ctr = jnp.concatenate([ctr_ref[0], ctr_ref[1]], axis=-1)
    h_ref[...] = jax.nn.relu(root + ctr + b_ref[...])


def _epi_tc(hh, contrib, b):
    return pl.pallas_call(
        _epi_body,
        grid=(NB,),
        in_specs=[
            pl.BlockSpec((1, 1, 2, BI, 128), lambda i: (i, R, 0, 0, 0)),
            pl.BlockSpec((2, BI, 128), lambda i: (0, i, 0)),
            pl.BlockSpec((1, H), lambda i: (0, 0)),
        ],
        out_specs=pl.BlockSpec((BI, H), lambda i: (i, 0)),
        out_shape=jax.ShapeDtypeStruct((N, H), jnp.float32),
    )(hh, contrib, b)


def _ss_body(b_ref, ss_ref):
    b = b_ref[...]
    g_ids = lax.broadcasted_iota(jnp.int32, (128, 1), 0)
    lt = (b < g_ids).astype(jnp.int32)
    ss_ref[...] = jnp.sum(lt, axis=1)


NPADB = 10240


def _ss_tc(batch_pad):
    return pl.pallas_call(
        _ss_body,
        in_specs=[pl.BlockSpec((1, NPADB), lambda: (0, 0))],
        out_specs=pl.BlockSpec((128,), lambda: (0,)),
        out_shape=jax.ShapeDtypeStruct((128,), jnp.int32),
    )(batch_pad)


def _pool_body(h_hbm, ss_hbm, out_hbm, ssv, rowbuf, acc):
    c = lax.axis_index("c")
    s = lax.axis_index("s")
    w = c * NS + s
    iota16 = jnp.arange(16, dtype=jnp.int32)
    ninf = jnp.full((16,), -jnp.inf, jnp.float32)
    for q in range(32):
        acc[q // 16, pl.ds((q % 16) * 16, 16)] = ninf
    pltpu.sync_copy(ss_hbm, ssv)

    for gg in range(2):
        g = 2 * w + gg
        ssl = ssv[pl.ds((g // 16) * 16, 16)]
        s0 = jnp.max(jnp.where(iota16 == g % 16, ssl, -1))
        gn = g + 1
        ssl2 = ssv[pl.ds((gn // 16) * 16, 16)]
        s1 = jnp.max(jnp.where(iota16 == gn % 16, ssl2, -1))
        ro0 = (s0 // 16) * 16
        nch = (s1 - ro0 + 15) // 16

        def chunk(ch, _):
            ro = jnp.minimum(ro0 + ch * 16, N - 16)
            pltpu.sync_copy(h_hbm.at[pl.ds(ro, 16)], rowbuf)

            def row(j, _):
                @pl.when(jnp.logical_and(ro + j >= s0, ro + j < s1))
                def _():
                    for q in range(16):
                        sl = pl.ds(q * 16, 16)
                        acc[gg, sl] = jnp.maximum(acc[gg, sl], rowbuf[j, sl])
                return 0
            lax.fori_loop(0, 16, row, 0)
            return 0
        lax.fori_loop(0, nch, chunk, 0)

    pltpu.sync_copy(acc, out_hbm.at[w])


def _pool_sc(h, ss):
    return pl.kernel(
        _pool_body,
        out_type=jax.ShapeDtypeStruct((NW, 2, H), jnp.float32),
        mesh=_MESH,
        scratch_types=[
            pltpu.VMEM((128,), jnp.int32),
            pltpu.VMEM((16, H), jnp.float32),
            pltpu.VMEM((2, H), jnp.float32),
        ],
        compiler_params=_SC_PARAMS,
    )(h, ss)


def _layer(x, src, dst, edge_type, inv_flat, w_cat, b):
    hh = _mm_all(x, w_cat)
    hh_flat = hh.reshape(NB * 9 * 2 * BI, 128)
    contrib = _agg_sc(src, dst, edge_type, inv_flat, hh_flat)
    return _epi_tc(hh, contrib, b.reshape(1, H))


def kernel(x, edge_index, edge_type, batch, Wr1, Wroot1, b1, Wr2, Wroot2, b2):
    src = edge_index[0]
    dst = edge_index[1]
    npad = EPAD - E
    dst_pad = jnp.concatenate([dst, jnp.full((npad,), N, jnp.int32)])
    type_pad = jnp.concatenate([edge_type, jnp.zeros((npad,), jnp.int32)])
    cnt_part = _cnt_sc(dst_pad, type_pad)
    inv_flat = _inv_tc(cnt_part)

    w_cat1 = jnp.concatenate([Wr1, Wroot1[None]], axis=0).astype(jnp.bfloat16)
    w_cat2 = jnp.concatenate([Wr2, Wroot2[None]], axis=0).astype(jnp.bfloat16)
    h1 = _layer(x, src, dst, edge_type, inv_flat, w_cat1, b1)
    h2 = _layer(h1, src, dst, edge_type, inv_flat, w_cat2, b2)

    batch_pad = jnp.concatenate(
        [batch, jnp.full((NPADB - N,), G, jnp.int32)]).reshape(1, NPADB)
    ss = _ss_tc(batch_pad)
    pooled = _pool_sc(h2, ss).reshape(G, H)
    return (h2, pooled)


# speedup vs baseline: 1.8546x; 1.0270x over previous
import functools

import jax
import jax.numpy as jnp
import numpy as np
from jax import lax
from jax.experimental import pallas as pl
from jax.experimental.pallas import tpu as pltpu
from jax.experimental.pallas import tpu_sc as plsc

N = 10000
E = 160000
F = 256
H = 256
R = 8
G = 64
BI = 400

NC, NS = 2, 16
NW = NC * NS
EPW = 5008
EPAD = NW * EPW
_MESH = plsc.VectorSubcoreMesh(core_axis_name="c", subcore_axis_name="s",
                               num_cores=NC, num_subcores=NS)
_SC_PARAMS = pltpu.CompilerParams(needs_layout_passes=False)


KROWS = 640
KDUMP = KROWS - 1


NK = KROWS * 128


def _cnt_body(dst_hbm, type_hbm, out_hbm, dbuf, tbuf, hist):
    c = lax.axis_index("c")
    s = lax.axis_index("s")
    w = c * NS + s

    def zslice(j, _):
        hist[pl.ds(j * 16, 16)] = jnp.zeros((16,), jnp.float32)
        return 0
    lax.fori_loop(0, NK // 16, zslice, 0)

    base = w * EPW
    pltpu.sync_copy(dst_hbm.at[pl.ds(base, EPW)], dbuf)
    pltpu.sync_copy(type_hbm.at[pl.ds(base, EPW)], tbuf)

    iota16 = jnp.arange(16, dtype=jnp.int32)
    ones16 = jnp.full((16,), 1.0, jnp.float32)

    def chunk(j, _):
        d16 = dbuf[pl.ds(j * 16, 16)]
        t16 = tbuf[pl.ds(j * 16, 16)]
        key = d16 * R + t16
        for l in range(16):
            plsc.addupdate_scatter(hist, [key], ones16, mask=iota16 == l)
        return 0

    lax.fori_loop(0, EPW // 16, chunk, 0)
    pltpu.sync_copy(hist, out_hbm.at[w])


def _cnt_sc(dst_pad, type_pad):
    return pl.kernel(
        _cnt_body,
        out_type=jax.ShapeDtypeStruct((NW, NK), jnp.float32),
        mesh=_MESH,
        scratch_types=[
            pltpu.VMEM((EPW,), jnp.int32),
            pltpu.VMEM((EPW,), jnp.int32),
            pltpu.VMEM((NK,), jnp.float32),
        ],
        compiler_params=_SC_PARAMS,
    )(dst_pad, type_pad)


def _inv_body(cnt_ref, inv_ref):
    tot = jnp.sum(cnt_ref[...], axis=0)
    inv_ref[...] = 1.0 / jnp.maximum(tot, 1.0)


def _inv_tc(cnt_part):
    blk = 8192
    return pl.pallas_call(
        _inv_body,
        grid=(NK // blk,),
        in_specs=[pl.BlockSpec((NW, blk), lambda i: (0, i))],
        out_specs=pl.BlockSpec((blk,), lambda i: (i,)),
        out_shape=jax.ShapeDtypeStruct((NK,), jnp.float32),
    )(cnt_part)


NB = N // BI

_t = np.arange(256)
_p = _t % 128
_PVEC = (_t // 128) * 128 + (_p // 32) * 32 + (_p % 2) * 16 + (_p % 32) // 2


def _mm_body(x_ref, w_ref, hh_ref):
    xb = x_ref[...].astype(jnp.bfloat16)
    for k in range(9):
        res = jnp.dot(xb, w_ref[k], preferred_element_type=jnp.float32)
        hh_ref[0, k, 0] = res[:, :128]
        hh_ref[0, k, 1] = res[:, 128:]


def _mm_all(x, w_cat):
    return pl.pallas_call(
        _mm_body,
        grid=(NB,),
        in_specs=[
            pl.BlockSpec((BI, F), lambda i: (i, 0)),
            pl.BlockSpec((9, F, H), lambda i: (0, 0, 0)),
        ],
        out_specs=pl.BlockSpec((1, 9, 2, BI, 128), lambda i: (i, 0, 0, 0, 0)),
        out_shape=jax.ShapeDtypeStruct((NB, 9, 2, BI, 128), jnp.float32),
    )(x, w_cat)


def _mmf_body(hh_ref, ctr_ref, b_ref, w_ref, hh2_ref):
    root = jnp.concatenate([hh_ref[0, 0, 0], hh_ref[0, 0, 1]], axis=-1)
    ctr = jnp.concatenate([ctr_ref[0], ctr_ref[1]], axis=-1)
    xb = jax.nn.relu(root + ctr + b_ref[...]).astype(jnp.bfloat16)
    for k in range(9):
        res = jnp.dot(xb, w_ref[k], preferred_element_type=jnp.float32)
        hh2_ref[0, k, 0] = res[:, :128]
        hh2_ref[0, k, 1] = res[:, 128:]


def _mm_fused(hh, contrib, b, w_cat):
    return pl.pallas_call(
        _mmf_body,
        grid=(NB,),
        in_specs=[
            pl.BlockSpec((1, 1, 2, BI, 128), lambda i: (i, R, 0, 0, 0)),
            pl.BlockSpec((2, BI, 128), lambda i: (0, i, 0)),
            pl.BlockSpec((1, H), lambda i: (0, 0)),
            pl.BlockSpec((9, F, H), lambda i: (0, 0, 0)),
        ],
        out_specs=pl.BlockSpec((1, 9, 2, BI, 128), lambda i: (i, 0, 0, 0, 0)),
        out_shape=jax.ShapeDtypeStruct((NB, 9, 2, BI, 128), jnp.float32),
    )(hh, contrib, b, w_cat)


EPT = E // NS
NSEG = 5
SEGE = EPT // NSEG
KCH = 80
NCH = SEGE // KCH


def _agg_body(src_hbm, dst_hbm, type_hbm, inv_hbm, hh_hbm, out_hbm,
              sbuf, tbuf, dbuf, gseg, dseg, kseg, nbuf, rows, zb,
              sem_g, sem_n, sem_s, acc_sh):
    c = lax.axis_index("c")
    s = lax.axis_index("s")
    iota16 = jnp.arange(16, dtype=jnp.int32)

    for j in range(16 * 8):
        zb[j // 8, pl.ds((j % 8) * 16, 16)] = jnp.zeros((16,), jnp.float32)
    for i in range(39):
        pltpu.sync_copy(zb, acc_sh.at[pl.ds(s * 624 + i * 16, 16)])
    @pl.when(s == NS - 1)
    def _():
        pltpu.sync_copy(zb, acc_sh.at[pl.ds(9984, 16)])
    plsc.subcore_barrier()

    base = s * EPT
    for seg in range(NSEG):
        sb = base + seg * SEGE
        pltpu.sync_copy(src_hbm.at[pl.ds(sb, SEGE)], sbuf)
        pltpu.sync_copy(dst_hbm.at[pl.ds(sb, SEGE)], dbuf)
        pltpu.sync_copy(type_hbm.at[pl.ds(sb, SEGE)], tbuf)

        def prep(q, _):
            cc = q // (KCH // 16)
            qq = q % (KCH // 16)
            sl = pl.ds(q * 16, 16)
            s16 = sbuf[sl]
            t16 = tbuf[sl]
            d16 = dbuf[sl]
            gi = (((s16 // BI) * 9 + t16) * 2 + c) * BI + s16 % BI
            gseg[cc, pl.ds(qq * 16, 16)] = gi
            dseg[cc, pl.ds(qq * 16, 16)] = d16
            kseg[cc, pl.ds(qq * 16, 16)] = d16 * R + t16
            return 0
        lax.fori_loop(0, SEGE // 16, prep, 0)

        pltpu.async_copy(hh_hbm.at[gseg.at[0]], rows.at[0], sem_g)
        pltpu.async_copy(inv_hbm.at[kseg.at[0]], nbuf.at[pl.ds(0, KCH)],
                         sem_n)

        def chunk(cc, _):
            @pl.when(cc > 0)
            def _():
                pltpu.make_async_copy(rows.at[(cc + 1) % 2],
                                      acc_sh.at[dseg.at[cc - 1]], sem_s).wait()
            @pl.when(cc < NCH - 1)
            def _():
                nb = ((cc + 1) % 2) * KCH
                pltpu.async_copy(hh_hbm.at[gseg.at[cc + 1]],
                                 rows.at[(cc + 1) % 2], sem_g)
                pltpu.async_copy(inv_hbm.at[kseg.at[cc + 1]],
                                 nbuf.at[pl.ds(nb, KCH)], sem_n)
            pltpu.make_async_copy(hh_hbm.at[gseg.at[cc]],
                                  rows.at[cc % 2], sem_g).wait()
            pltpu.make_async_copy(inv_hbm.at[kseg.at[cc]],
                                  nbuf.at[pl.ds((cc % 2) * KCH, KCH)],
                                  sem_n).wait()

            def scale(e, _):
                sp = plsc.load_gather(nbuf, [jnp.full((16,), 0, jnp.int32)
                                             + ((cc % 2) * KCH + e)])
                for q in range(8):
                    sl = pl.ds(q * 16, 16)
                    rows[cc % 2, e, sl] = rows[cc % 2, e, sl] * sp
                return 0
            lax.fori_loop(0, KCH, scale, 0)
            pltpu.async_copy(rows.at[cc % 2], acc_sh.at[dseg.at[cc]], sem_s,
                             add=True)
            return 0
        lax.fori_loop(0, NCH, chunk, 0)
        pltpu.make_async_copy(rows.at[(NCH - 1) % 2],
                              acc_sh.at[dseg.at[NCH - 1]], sem_s).wait()

    plsc.subcore_barrier()
    pltpu.sync_copy(acc_sh.at[pl.ds(s * 624, 624)],
                    out_hbm.at[c, pl.ds(s * 624, 624)])
    @pl.when(s == NS - 1)
    def _():
        pltpu.sync_copy(acc_sh.at[pl.ds(9984, 16)],
                        out_hbm.at[c, pl.ds(9984, 16)])


def _agg_sc(src, dst, etype, inv_flat, hh_flat):
    return pl.kernel(
        _agg_body,
        out_type=jax.ShapeDtypeStruct((NC, N, 128), jnp.float32),
        mesh=_MESH,
        scratch_types=[
            pltpu.VMEM((SEGE,), jnp.int32),
            pltpu.VMEM((SEGE,), jnp.int32),
            pltpu.VMEM((SEGE,), jnp.int32),
            pltpu.VMEM((NCH, KCH), jnp.int32),
            pltpu.VMEM((NCH, KCH), jnp.int32),
            pltpu.VMEM((NCH, KCH), jnp.int32),
            pltpu.VMEM((2 * KCH,), jnp.float32),
            pltpu.VMEM((2, KCH, 128), jnp.float32),
            pltpu.VMEM((16, 128), jnp.float32),
            pltpu.SemaphoreType.DMA,
            pltpu.SemaphoreType.DMA,
            pltpu.SemaphoreType.DMA,
            pltpu.VMEM_SHARED((N, 128), jnp.float32),
        ],
        compiler_params=_SC_PARAMS,
    )(src, dst, etype, inv_flat, hh_flat)


def _epi_body(hh_ref, ctr_ref, b_ref, h_ref):
    root = jnp.concatenate([hh_ref[0, 0, 0], hh_ref[0, 0, 1]],
                           axis=-1).astype(jnp.float32)
    ctr = jnp.concatenate([ctr_ref[0], ctr_ref[1]], axis=-1)
    h_ref[...] = jax.nn.relu(root + ctr + b_ref[...])


def _epi_tc(hh, contrib, b):
    return pl.pallas_call(
        _epi_body,
        grid=(NB,),
        in_specs=[
            pl.BlockSpec((1, 1, 2, BI, 128), lambda i: (i, R, 0, 0, 0)),
            pl.BlockSpec((2, BI, 128), lambda i: (0, i, 0)),
            pl.BlockSpec((1, H), lambda i: (0, 0)),
        ],
        out_specs=pl.BlockSpec((BI, H), lambda i: (i, 0)),
        out_shape=jax.ShapeDtypeStruct((N, H), jnp.float32),
    )(hh, contrib, b)


def _ss_body(b_ref, ss_ref):
    b = b_ref[...]
    g_ids = lax.broadcasted_iota(jnp.int32, (128, 1), 0)
    lt = (b < g_ids).astype(jnp.int32)
    ss_ref[...] = jnp.sum(lt, axis=1)


NPADB = 10240


def _ss_tc(batch_pad):
    return pl.pallas_call(
        _ss_body,
        in_specs=[pl.BlockSpec((1, NPADB), lambda: (0, 0))],
        out_specs=pl.BlockSpec((128,), lambda: (0,)),
        out_shape=jax.ShapeDtypeStruct((128,), jnp.int32),
    )(batch_pad)


def _pool_body(h_hbm, ss_hbm, out_hbm, ssv, rowbuf, acc):
    c = lax.axis_index("c")
    s = lax.axis_index("s")
    w = c * NS + s
    iota16 = jnp.arange(16, dtype=jnp.int32)
    ninf = jnp.full((16,), -jnp.inf, jnp.float32)
    for q in range(32):
        acc[q // 16, pl.ds((q % 16) * 16, 16)] = ninf
    pltpu.sync_copy(ss_hbm, ssv)

    for gg in range(2):
        g = 2 * w + gg
        ssl = ssv[pl.ds((g // 16) * 16, 16)]
        s0 = jnp.max(jnp.where(iota16 == g % 16, ssl, -1))
        gn = g + 1
        ssl2 = ssv[pl.ds((gn // 16) * 16, 16)]
        s1 = jnp.max(jnp.where(iota16 == gn % 16, ssl2, -1))
        ro0 = (s0 // 16) * 16
        nch = (s1 - ro0 + 15) // 16

        def chunk(ch, _):
            ro = jnp.minimum(ro0 + ch * 16, N - 16)
            pltpu.sync_copy(h_hbm.at[pl.ds(ro, 16)], rowbuf)

            def row(j, _):
                @pl.when(jnp.logical_and(ro + j >= s0, ro + j < s1))
                def _():
                    for q in range(16):
                        sl = pl.ds(q * 16, 16)
                        acc[gg, sl] = jnp.maximum(acc[gg, sl], rowbuf[j, sl])
                return 0
            lax.fori_loop(0, 16, row, 0)
            return 0
        lax.fori_loop(0, nch, chunk, 0)

    pltpu.sync_copy(acc, out_hbm.at[w])


def _pool_sc(h, ss):
    return pl.kernel(
        _pool_body,
        out_type=jax.ShapeDtypeStruct((NW, 2, H), jnp.float32),
        mesh=_MESH,
        scratch_types=[
            pltpu.VMEM((128,), jnp.int32),
            pltpu.VMEM((16, H), jnp.float32),
            pltpu.VMEM((2, H), jnp.float32),
        ],
        compiler_params=_SC_PARAMS,
    )(h, ss)


def _flat(hh):
    return hh.reshape(NB * 9 * 2 * BI, 128)


def kernel(x, edge_index, edge_type, batch, Wr1, Wroot1, b1, Wr2, Wroot2, b2):
    src = edge_index[0]
    dst = edge_index[1]
    npad = EPAD - E
    dst_pad = jnp.concatenate([dst, jnp.full((npad,), N, jnp.int32)])
    type_pad = jnp.concatenate([edge_type, jnp.zeros((npad,), jnp.int32)])
    cnt_part = _cnt_sc(dst_pad, type_pad)
    inv_flat = _inv_tc(cnt_part)

    w_cat1 = jnp.concatenate([Wr1, Wroot1[None]], axis=0).astype(jnp.bfloat16)
    w_cat2 = jnp.concatenate([Wr2, Wroot2[None]], axis=0).astype(jnp.bfloat16)
    hh1 = _mm_all(x, w_cat1)
    c1 = _agg_sc(src, dst, edge_type, inv_flat, _flat(hh1))
    hh2 = _mm_fused(hh1, c1, b1.reshape(1, H), w_cat2)
    c2 = _agg_sc(src, dst, edge_type, inv_flat, _flat(hh2))
    h2 = _epi_tc(hh2, c2, b2.reshape(1, H))

    batch_pad = jnp.concatenate(
        [batch, jnp.full((NPADB - N,), G, jnp.int32)]).reshape(1, NPADB)
    ss = _ss_tc(batch_pad)
    pooled = _pool_sc(h2, ss).reshape(G, H)
    return (h2, pooled)


# speedup vs baseline: 1.8999x; 1.0245x over previous
import functools

import jax
import jax.numpy as jnp
import numpy as np
from jax import lax
from jax.experimental import pallas as pl
from jax.experimental.pallas import tpu as pltpu
from jax.experimental.pallas import tpu_sc as plsc

N = 10000
E = 160000
F = 256
H = 256
R = 8
G = 64
BI = 400

NC, NS = 2, 16
NW = NC * NS
EPW = 5008
EPAD = NW * EPW
_MESH = plsc.VectorSubcoreMesh(core_axis_name="c", subcore_axis_name="s",
                               num_cores=NC, num_subcores=NS)
_SC_PARAMS = pltpu.CompilerParams(needs_layout_passes=False)


KROWS = 640
KDUMP = KROWS - 1


NK = KROWS * 128


def _cnt_body(dst_hbm, type_hbm, out_hbm, dbuf, tbuf, hist):
    c = lax.axis_index("c")
    s = lax.axis_index("s")
    w = c * NS + s

    def zslice(j, _):
        hist[pl.ds(j * 16, 16)] = jnp.zeros((16,), jnp.float32)
        return 0
    lax.fori_loop(0, NK // 16, zslice, 0)

    base = w * EPW
    pltpu.sync_copy(dst_hbm.at[pl.ds(base, EPW)], dbuf)
    pltpu.sync_copy(type_hbm.at[pl.ds(base, EPW)], tbuf)

    iota16 = jnp.arange(16, dtype=jnp.int32)
    ones16 = jnp.full((16,), 1.0, jnp.float32)

    def chunk(j, _):
        d16 = dbuf[pl.ds(j * 16, 16)]
        t16 = tbuf[pl.ds(j * 16, 16)]
        key = d16 * R + t16
        for l in range(16):
            plsc.addupdate_scatter(hist, [key], ones16, mask=iota16 == l)
        return 0

    lax.fori_loop(0, EPW // 16, chunk, 0)
    pltpu.sync_copy(hist, out_hbm.at[w])


def _cnt_sc(dst_pad, type_pad):
    return pl.kernel(
        _cnt_body,
        out_type=jax.ShapeDtypeStruct((NW, NK), jnp.float32),
        mesh=_MESH,
        scratch_types=[
            pltpu.VMEM((EPW,), jnp.int32),
            pltpu.VMEM((EPW,), jnp.int32),
            pltpu.VMEM((NK,), jnp.float32),
        ],
        compiler_params=_SC_PARAMS,
    )(dst_pad, type_pad)


def _inv_body(cnt_ref, inv_ref):
    tot = jnp.sum(cnt_ref[...], axis=0)
    inv_ref[...] = 1.0 / jnp.maximum(tot, 1.0)


def _inv_tc(cnt_part):
    blk = 8192
    return pl.pallas_call(
        _inv_body,
        grid=(NK // blk,),
        in_specs=[pl.BlockSpec((NW, blk), lambda i: (0, i))],
        out_specs=pl.BlockSpec((blk,), lambda i: (i,)),
        out_shape=jax.ShapeDtypeStruct((NK,), jnp.float32),
    )(cnt_part)


NB = N // BI

_t = np.arange(256)
_p = _t % 128
_PVEC = (_t // 128) * 128 + (_p // 32) * 32 + (_p % 2) * 16 + (_p % 32) // 2


def _mm_body(x_ref, w_ref, hh_ref):
    xb = x_ref[...].astype(jnp.bfloat16)
    for k in range(9):
        res = jnp.dot(xb, w_ref[k], preferred_element_type=jnp.float32)
        hh_ref[0, k, 0] = res[:, :128]
        hh_ref[0, k, 1] = res[:, 128:]


def _mm_all(x, w_cat):
    return pl.pallas_call(
        _mm_body,
        grid=(NB,),
        in_specs=[
            pl.BlockSpec((BI, F), lambda i: (i, 0)),
            pl.BlockSpec((9, F, H), lambda i: (0, 0, 0)),
        ],
        out_specs=pl.BlockSpec((1, 9, 2, BI, 128), lambda i: (i, 0, 0, 0, 0)),
        out_shape=jax.ShapeDtypeStruct((NB, 9, 2, BI, 128), jnp.float32),
    )(x, w_cat)


def _mmf_body(hh_ref, ctr_ref, b_ref, w_ref, hh2_ref):
    root = jnp.concatenate([hh_ref[0, 0, 0], hh_ref[0, 0, 1]], axis=-1)
    ctr = jnp.concatenate([ctr_ref[0], ctr_ref[1]], axis=-1)
    xb = jax.nn.relu(root + ctr + b_ref[...]).astype(jnp.bfloat16)
    for k in range(9):
        res = jnp.dot(xb, w_ref[k], preferred_element_type=jnp.float32)
        hh2_ref[0, k, 0] = res[:, :128]
        hh2_ref[0, k, 1] = res[:, 128:]


def _mm_fused(hh, contrib, b, w_cat):
    return pl.pallas_call(
        _mmf_body,
        grid=(NB,),
        in_specs=[
            pl.BlockSpec((1, 1, 2, BI, 128), lambda i: (i, R, 0, 0, 0)),
            pl.BlockSpec((2, BI, 128), lambda i: (0, i, 0)),
            pl.BlockSpec((1, H), lambda i: (0, 0)),
            pl.BlockSpec((9, F, H), lambda i: (0, 0, 0)),
        ],
        out_specs=pl.BlockSpec((1, 9, 2, BI, 128), lambda i: (i, 0, 0, 0, 0)),
        out_shape=jax.ShapeDtypeStruct((NB, 9, 2, BI, 128), jnp.float32),
    )(hh, contrib, b, w_cat)


EPT = E // NS
NSEG = 5
SEGE = EPT // NSEG
KCH = 80
NCH = SEGE // KCH


def _agg_body(src_hbm, dst_hbm, type_hbm, inv_hbm, hh_hbm, out_hbm,
              sbuf, tbuf, dbuf, gseg, dseg, kseg, nbuf, rows, zb,
              sem_g, sem_n, sem_s, acc_sh):
    c = lax.axis_index("c")
    s = lax.axis_index("s")
    iota16 = jnp.arange(16, dtype=jnp.int32)

    for j in range(16 * 8):
        zb[j // 8, pl.ds((j % 8) * 16, 16)] = jnp.zeros((16,), jnp.float32)
    for i in range(39):
        pltpu.sync_copy(zb, acc_sh.at[pl.ds(s * 624 + i * 16, 16)])
    @pl.when(s == NS - 1)
    def _():
        pltpu.sync_copy(zb, acc_sh.at[pl.ds(9984, 16)])
    plsc.subcore_barrier()

    base = s * EPT
    for seg in range(NSEG):
        sb = base + seg * SEGE
        pltpu.sync_copy(src_hbm.at[pl.ds(sb, SEGE)], sbuf)
        pltpu.sync_copy(dst_hbm.at[pl.ds(sb, SEGE)], dbuf)
        pltpu.sync_copy(type_hbm.at[pl.ds(sb, SEGE)], tbuf)

        def prep(q, _):
            cc = q // (KCH // 16)
            qq = q % (KCH // 16)
            sl = pl.ds(q * 16, 16)
            s16 = sbuf[sl]
            t16 = tbuf[sl]
            d16 = dbuf[sl]
            gi = (((s16 // BI) * 9 + t16) * 2 + c) * BI + s16 % BI
            gseg[cc, pl.ds(qq * 16, 16)] = gi
            dseg[cc, pl.ds(qq * 16, 16)] = d16
            kseg[cc, pl.ds(qq * 16, 16)] = d16 * R + t16
            return 0
        lax.fori_loop(0, SEGE // 16, prep, 0)

        pltpu.async_copy(hh_hbm.at[gseg.at[0]], rows.at[0], sem_g)
        pltpu.async_copy(inv_hbm.at[kseg.at[0]], nbuf.at[pl.ds(0, KCH)],
                         sem_n)

        def chunk(cc, _):
            @pl.when(cc > 0)
            def _():
                pltpu.make_async_copy(rows.at[(cc + 1) % 2],
                                      acc_sh.at[dseg.at[cc - 1]], sem_s).wait()
            @pl.when(cc < NCH - 1)
            def _():
                nb = ((cc + 1) % 2) * KCH
                pltpu.async_copy(hh_hbm.at[gseg.at[cc + 1]],
                                 rows.at[(cc + 1) % 2], sem_g)
                pltpu.async_copy(inv_hbm.at[kseg.at[cc + 1]],
                                 nbuf.at[pl.ds(nb, KCH)], sem_n)
            pltpu.make_async_copy(hh_hbm.at[gseg.at[cc]],
                                  rows.at[cc % 2], sem_g).wait()
            pltpu.make_async_copy(inv_hbm.at[kseg.at[cc]],
                                  nbuf.at[pl.ds((cc % 2) * KCH, KCH)],
                                  sem_n).wait()

            def scale(e, _):
                sp = plsc.load_gather(nbuf, [jnp.full((16,), 0, jnp.int32)
                                             + ((cc % 2) * KCH + e)])
                for q in range(8):
                    sl = pl.ds(q * 16, 16)
                    rows[cc % 2, e, sl] = rows[cc % 2, e, sl] * sp
                return 0
            lax.fori_loop(0, KCH, scale, 0)
            pltpu.async_copy(rows.at[cc % 2], acc_sh.at[dseg.at[cc]], sem_s,
                             add=True)
            return 0
        lax.fori_loop(0, NCH, chunk, 0)
        pltpu.make_async_copy(rows.at[(NCH - 1) % 2],
                              acc_sh.at[dseg.at[NCH - 1]], sem_s).wait()

    plsc.subcore_barrier()
    pltpu.sync_copy(acc_sh.at[pl.ds(s * 624, 624)],
                    out_hbm.at[c, pl.ds(s * 624, 624)])
    @pl.when(s == NS - 1)
    def _():
        pltpu.sync_copy(acc_sh.at[pl.ds(9984, 16)],
                        out_hbm.at[c, pl.ds(9984, 16)])


def _agg_sc(src, dst, etype, inv_flat, hh_flat):
    return pl.kernel(
        _agg_body,
        out_type=jax.ShapeDtypeStruct((NC, N, 128), jnp.float32),
        mesh=_MESH,
        scratch_types=[
            pltpu.VMEM((SEGE,), jnp.int32),
            pltpu.VMEM((SEGE,), jnp.int32),
            pltpu.VMEM((SEGE,), jnp.int32),
            pltpu.VMEM((NCH, KCH), jnp.int32),
            pltpu.VMEM((NCH, KCH), jnp.int32),
            pltpu.VMEM((NCH, KCH), jnp.int32),
            pltpu.VMEM((2 * KCH,), jnp.float32),
            pltpu.VMEM((2, KCH, 128), jnp.float32),
            pltpu.VMEM((16, 128), jnp.float32),
            pltpu.SemaphoreType.DMA,
            pltpu.SemaphoreType.DMA,
            pltpu.SemaphoreType.DMA,
            pltpu.VMEM_SHARED((N, 128), jnp.float32),
        ],
        compiler_params=_SC_PARAMS,
    )(src, dst, etype, inv_flat, hh_flat)


def _epi_body(hh_ref, ctr_ref, b_ref, h_ref):
    root = jnp.concatenate([hh_ref[0, 0, 0], hh_ref[0, 0, 1]],
                           axis=-1).astype(jnp.float32)
    ctr = jnp.concatenate([ctr_ref[0], ctr_ref[1]], axis=-1)
    h_ref[...] = jax.nn.relu(root + ctr + b_ref[...])


def _epi_tc(hh, contrib, b):
    return pl.pallas_call(
        _epi_body,
        grid=(NB,),
        in_specs=[
            pl.BlockSpec((1, 1, 2, BI, 128), lambda i: (i, R, 0, 0, 0)),
            pl.BlockSpec((2, BI, 128), lambda i: (0, i, 0)),
            pl.BlockSpec((1, H), lambda i: (0, 0)),
        ],
        out_specs=pl.BlockSpec((BI, H), lambda i: (i, 0)),
        out_shape=jax.ShapeDtypeStruct((N, H), jnp.float32),
    )(hh, contrib, b)


def _ss_body(b_ref, ss_ref):
    b = b_ref[...]
    g_ids = lax.broadcasted_iota(jnp.int32, (128, 1), 0)
    lt = (b < g_ids).astype(jnp.int32)
    ss_ref[...] = jnp.sum(lt, axis=1)


NPADB = 10240


def _ss_tc(batch_pad):
    return pl.pallas_call(
        _ss_body,
        in_specs=[pl.BlockSpec((1, NPADB), lambda: (0, 0))],
        out_specs=pl.BlockSpec((128,), lambda: (0,)),
        out_shape=jax.ShapeDtypeStruct((128,), jnp.int32),
    )(batch_pad)


def _pool_body(h_hbm, ss_hbm, out_hbm, ssv, rowbuf, acc, sem):
    c = lax.axis_index("c")
    s = lax.axis_index("s")
    w = c * NS + s
    iota16 = jnp.arange(16, dtype=jnp.int32)
    ninf = jnp.full((16,), -jnp.inf, jnp.float32)
    for q in range(32):
        acc[q // 16, pl.ds((q % 16) * 16, 16)] = ninf
    pltpu.sync_copy(ss_hbm, ssv)

    for gg in range(2):
        g = 2 * w + gg
        ssl = ssv[pl.ds((g // 16) * 16, 16)]
        s0 = jnp.max(jnp.where(iota16 == g % 16, ssl, -1))
        gn = g + 1
        ssl2 = ssv[pl.ds((gn // 16) * 16, 16)]
        s1 = jnp.max(jnp.where(iota16 == gn % 16, ssl2, -1))
        ro0 = (s0 // 16) * 16
        nch = (s1 - ro0 + 15) // 16

        def _ro(ch):
            return jnp.minimum(ro0 + ch * 16, N - 16)

        @pl.when(nch > 0)
        def _():
            pltpu.async_copy(h_hbm.at[pl.ds(_ro(0), 16)], rowbuf.at[0], sem)

        def chunk(ch, _):
            @pl.when(ch + 1 < nch)
            def _():
                pltpu.async_copy(h_hbm.at[pl.ds(_ro(ch + 1), 16)],
                                 rowbuf.at[(ch + 1) % 2], sem)
            ro = _ro(ch)
            pltpu.make_async_copy(h_hbm.at[pl.ds(ro, 16)],
                                  rowbuf.at[ch % 2], sem).wait()

            def row(j, _):
                @pl.when(jnp.logical_and(ro + j >= s0, ro + j < s1))
                def _():
                    for q in range(16):
                        sl = pl.ds(q * 16, 16)
                        acc[gg, sl] = jnp.maximum(acc[gg, sl],
                                                  rowbuf[ch % 2, j, sl])
                return 0
            lax.fori_loop(0, 16, row, 0)
            return 0
        lax.fori_loop(0, nch, chunk, 0)

    pltpu.sync_copy(acc, out_hbm.at[w])


def _pool_sc(h, ss):
    return pl.kernel(
        _pool_body,
        out_type=jax.ShapeDtypeStruct((NW, 2, H), jnp.float32),
        mesh=_MESH,
        scratch_types=[
            pltpu.VMEM((128,), jnp.int32),
            pltpu.VMEM((2, 16, H), jnp.float32),
            pltpu.VMEM((2, H), jnp.float32),
            pltpu.SemaphoreType.DMA,
        ],
        compiler_params=_SC_PARAMS,
    )(h, ss)


def _flat(hh):
    return hh.reshape(NB * 9 * 2 * BI, 128)


def kernel(x, edge_index, edge_type, batch, Wr1, Wroot1, b1, Wr2, Wroot2, b2):
    src = edge_index[0]
    dst = edge_index[1]
    npad = EPAD - E
    dst_pad = jnp.concatenate([dst, jnp.full((npad,), N, jnp.int32)])
    type_pad = jnp.concatenate([edge_type, jnp.zeros((npad,), jnp.int32)])
    cnt_part = _cnt_sc(dst_pad, type_pad)
    inv_flat = _inv_tc(cnt_part)

    w_cat1 = jnp.concatenate([Wr1, Wroot1[None]], axis=0).astype(jnp.bfloat16)
    w_cat2 = jnp.concatenate([Wr2, Wroot2[None]], axis=0).astype(jnp.bfloat16)
    hh1 = _mm_all(x, w_cat1)
    c1 = _agg_sc(src, dst, edge_type, inv_flat, _flat(hh1))
    hh2 = _mm_fused(hh1, c1, b1.reshape(1, H), w_cat2)
    c2 = _agg_sc(src, dst, edge_type, inv_flat, _flat(hh2))
    h2 = _epi_tc(hh2, c2, b2.reshape(1, H))

    batch_pad = jnp.concatenate(
        [batch, jnp.full((NPADB - N,), G, jnp.int32)]).reshape(1, NPADB)
    ss = _ss_tc(batch_pad)
    pooled = _pool_sc(h2, ss).reshape(G, H)
    return (h2, pooled)
